# baseline (speedup 1.0000x reference)
import functools

import jax
import jax.numpy as jnp
import numpy as np
from jax.experimental import pallas as pl

_ANCHORS = np.array([[1.146, 1.621, 3.88],
                     [1.52, 1.93, 5.08],
                     [1.73, 2.58, 10.1]], dtype=np.float32)
_NC = 3
_NB = 9
_C = _NB + _NC
_G = 76
_K = _G * _G
_STRIDE = 8.0

_SCALES = np.ones((3, _C, 1), dtype=np.float32)
_SCALES[:, 0, 0] = _STRIDE
_SCALES[:, 1, 0] = _STRIDE
_SCALES[:, 3:6, 0] = _ANCHORS


def _decode_kernel(x_ref, o_ref):
    a = pl.program_id(0) % 3
    v = x_ref[0]
    s = jax.nn.sigmoid(v)
    e = jnp.minimum(jnp.exp(v), 1000.0)
    rows = jax.lax.broadcasted_iota(jnp.int32, (_C, _K), 0)
    sig_mask = (rows <= 2) | (rows >= 8)
    exp_mask = (rows >= 3) & (rows <= 5)
    base = jnp.where(sig_mask, s, jnp.where(exp_mask, e, v))
    scale = jnp.where(rows <= 1, _STRIDE, 1.0)
    for j, r in enumerate((3, 4, 5)):
        aval = jnp.where(a == 0, float(_ANCHORS[0, j]),
                         jnp.where(a == 1, float(_ANCHORS[1, j]),
                                   float(_ANCHORS[2, j])))
        scale = jnp.where(rows == r, aval, scale)
    k = jax.lax.broadcasted_iota(jnp.int32, (_C, _K), 1)
    gx = (k % _G).astype(jnp.float32) * _STRIDE
    gy = (k // _G).astype(jnp.float32) * _STRIDE
    add = jnp.where(rows == 0, gx, jnp.where(rows == 1, gy, 0.0))
    res = base * scale + add
    o_ref[0] = res.T


@jax.jit
def kernel(x):
    B = x.shape[0]
    nA = 3
    x2 = x.reshape(B * nA, _C, _K)
    out = pl.pallas_call(
        _decode_kernel,
        grid=(B * nA,),
        in_specs=[pl.BlockSpec((1, _C, _K), lambda i: (i, 0, 0))],
        out_specs=pl.BlockSpec((1, _K, _C), lambda i: (i, 0, 0)),
        out_shape=jax.ShapeDtypeStruct((B * nA, _K, _C), jnp.float32),
    )(x2)
    return out.reshape(B, nA * _K, _C)
